# baseline (speedup 1.0000x reference)
import jax
import jax.numpy as jnp
from jax import lax
from jax.experimental import pallas as pl
from jax.experimental.pallas import tpu as pltpu
from jax.experimental.pallas import tpu_sc as plsc

N = 10000
E = 320000
D = 128
R = 4
H1 = 16
H2 = 32
C = 8
G = 64

RN = R * N
PAD_ROWS = 64
RNP = RN + PAD_ROWS
NSUB = 16
NCORE = 2
NW = NSUB * NCORE
EB = 128
EPW = 10240
NBLK = EPW // EB
CW = 8
EPAD = NW * EPW
SUB_ROWS = RNP // NSUB
NB = 2000
NGRID = N // NB


def _y1_body(x_ref, w_ref, y_ref):
    x = x_ref[...]
    for r in range(R):
        y_ref[r] = jnp.dot(x, w_ref[r], preferred_element_type=jnp.float32)


def _relation_matmul(x, W1):
    return pl.pallas_call(
        _y1_body,
        grid=(NGRID,),
        in_specs=[pl.BlockSpec((NB, D), lambda i: (i, 0)),
                  pl.BlockSpec((R, D, H1), lambda i: (0, 0, 0))],
        out_specs=pl.BlockSpec((R, NB, H1), lambda i: (0, i, 0)),
        out_shape=jax.ShapeDtypeStruct((R, N, H1), jnp.float32),
    )(x, W1)


def _make_edge_scatter(width, do_cnt, nbuf):
    mesh = plsc.VectorSubcoreMesh(core_axis_name="c", subcore_axis_name="s")
    out_type = [jax.ShapeDtypeStruct((NCORE, RNP, width), jnp.float32)]
    scratch = [
        pltpu.VMEM_SHARED((RNP, width), jnp.float32),
        pltpu.VMEM((NBLK, EB), jnp.int32),
        pltpu.VMEM((NBLK, EB), jnp.int32),
    ] + [pltpu.VMEM((EB, width), jnp.float32) for _ in range(nbuf)] + [
        pltpu.SemaphoreType.DMA((nbuf,)),
        pltpu.SemaphoreType.DMA((nbuf,)),
    ]
    if do_cnt:
        out_type.append(jax.ShapeDtypeStruct((NCORE, RNP, CW), jnp.float32))
        scratch += [
            pltpu.VMEM_SHARED((RNP, CW), jnp.float32),
            pltpu.VMEM((EB, CW), jnp.float32),
            pltpu.SemaphoreType.DMA((nbuf,)),
        ]

    def body(y_hbm, g_hbm, s_hbm, z_hbm, *rest):
        if do_cnt:
            (z8_hbm, ones_hbm, out_acc, out_cnt,
             acc_sh, g_all, s_all, *rbuf, gsem, ssem,
             cnt_sh, ones_v, csem) = rest
        else:
            (out_acc, acc_sh, g_all, s_all, *rbuf, gsem, ssem) = rest
        cid = lax.axis_index("c")
        sid = lax.axis_index("s")
        wid = cid * NSUB + sid
        rows = pl.ds(sid * SUB_ROWS, SUB_ROWS)
        pltpu.sync_copy(g_hbm.at[wid], g_all)
        pltpu.sync_copy(s_hbm.at[wid], s_all)
        pltpu.sync_copy(z_hbm.at[rows], acc_sh.at[rows])
        if do_cnt:
            pltpu.sync_copy(z8_hbm.at[rows], cnt_sh.at[rows])
            pltpu.sync_copy(ones_hbm, ones_v)
        plsc.subcore_barrier()

        for k in range(nbuf):
            pltpu.async_copy(y_hbm.at[g_all.at[k]], rbuf[k], gsem.at[k])

        def step(j, carry):
            for k in range(nbuf):
                b = j * nbuf + k
                pltpu.make_async_copy(y_hbm.at[g_all.at[b]], rbuf[k],
                                      gsem.at[k]).wait()
                if do_cnt:
                    pltpu.async_copy(ones_v, cnt_sh.at[s_all.at[b]],
                                     csem.at[k], add=True)
                pltpu.async_copy(rbuf[k], acc_sh.at[s_all.at[b]],
                                 ssem.at[k], add=True)
            for k in range(nbuf):
                b = j * nbuf + k
                pltpu.make_async_copy(rbuf[k], acc_sh.at[s_all.at[b]],
                                      ssem.at[k]).wait()
                if do_cnt:
                    pltpu.make_async_copy(ones_v, cnt_sh.at[s_all.at[b]],
                                          csem.at[k]).wait()

                @pl.when(j < NBLK // nbuf - 1)
                def _(k=k, b=b):
                    pltpu.async_copy(y_hbm.at[g_all.at[b + nbuf]], rbuf[k],
                                     gsem.at[k])
            return carry

        lax.fori_loop(0, NBLK // nbuf, step, 0)
        plsc.subcore_barrier()
        pltpu.sync_copy(acc_sh.at[rows], out_acc.at[cid, rows])
        if do_cnt:
            pltpu.sync_copy(cnt_sh.at[rows], out_cnt.at[cid, rows])

    return pl.kernel(body, out_type=out_type, mesh=mesh, scratch_types=scratch,
                     compiler_params=pltpu.CompilerParams(
                         use_tc_tiling_on_sc=False))


_scatter1 = _make_edge_scatter(H1, True, 8)
_scatter2 = _make_edge_scatter(H2, False, 4)


def _mid_body(x_ref, a0, a1, a2, a3, c0, c1, c2, c3, r1_ref, b1_ref, w2_ref,
              r2_ref, b2_ref, y2_ref, z2_ref):
    msg = jnp.zeros((NB, H1), jnp.float32)
    for a_ref, c_ref in ((a0, c0), (a1, c1), (a2, c2), (a3, c3)):
        a = a_ref[0] + a_ref[1]
        c8 = c_ref[0] + c_ref[1]
        c = jnp.concatenate([c8, c8], axis=-1)
        msg = msg + a / jnp.maximum(c, 1.0)
    h1 = jax.nn.relu(
        jnp.dot(x_ref[...], r1_ref[...], preferred_element_type=jnp.float32)
        + b1_ref[...] + msg)
    for r in range(R):
        y2_ref[r] = jnp.dot(h1, w2_ref[r], preferred_element_type=jnp.float32)
    z2_ref[...] = (jnp.dot(h1, r2_ref[...], preferred_element_type=jnp.float32)
                   + b2_ref[...])


def _acc_specs(width):
    return [pl.BlockSpec((NCORE, NB, width), lambda i, r=r: (0, r * NGRID + i, 0))
            for r in range(R)]


def _mid_layer(x, acc1, cnt, root1, b1, W2, root2, b2):
    return pl.pallas_call(
        _mid_body,
        grid=(NGRID,),
        in_specs=[pl.BlockSpec((NB, D), lambda i: (i, 0))]
        + _acc_specs(H1) + _acc_specs(CW) + [
            pl.BlockSpec((D, H1), lambda i: (0, 0)),
            pl.BlockSpec((1, H1), lambda i: (0, 0)),
            pl.BlockSpec((R, H1, H2), lambda i: (0, 0, 0)),
            pl.BlockSpec((H1, H2), lambda i: (0, 0)),
            pl.BlockSpec((1, H2), lambda i: (0, 0)),
        ],
        out_specs=[
            pl.BlockSpec((R, NB, H2), lambda i: (0, i, 0)),
            pl.BlockSpec((NB, H2), lambda i: (i, 0)),
        ],
        out_shape=[
            jax.ShapeDtypeStruct((R, N, H2), jnp.float32),
            jax.ShapeDtypeStruct((N, H2), jnp.float32),
        ],
    )(x, *([acc1] * R), *([cnt] * R), root1, b1, W2, root2, b2)


GROUPS = NB // 4


def _final_body(z_ref, a0, a1, a2, a3, c0, c1, c2, c3, b_ref, wl1_ref,
                bl1_ref, wl2_ref, bl2_ref, o_ref, p_ref):
    i = pl.program_id(0)
    msg = jnp.zeros((NB, H2), jnp.float32)
    for a_ref, c_ref in ((a0, c0), (a1, c1), (a2, c2), (a3, c3)):
        a = a_ref[0] + a_ref[1]
        c8 = c_ref[0] + c_ref[1]
        c32 = jnp.concatenate([c8, c8, c8, c8], axis=-1)
        msg = msg + a / jnp.maximum(c32, 1.0)
    h2 = jax.nn.relu(z_ref[...] + msg)

    @pl.when(i == 0)
    def _():
        p_ref[...] = jnp.zeros((G, H2), jnp.float32)

    b = b_ref[...]
    gmin = b_ref[0, 0]
    gmax = b_ref[NB - 1, 0]

    def pool(g, carry):
        v = jnp.max(jnp.where(b == g, h2, 0.0), axis=0, keepdims=True)
        p_ref[pl.ds(g, 1), :] = jnp.maximum(p_ref[pl.ds(g, 1), :], v)
        return carry

    lax.fori_loop(gmin, gmax + 1, pool, 0)

    @pl.when(i == NGRID - 1)
    def _():
        p = p_ref[...]
        hh = jax.nn.relu(
            jnp.dot(p, wl1_ref[...], preferred_element_type=jnp.float32)
            + bl1_ref[...])
        o_ref[...] = (jnp.dot(hh, wl2_ref[...], preferred_element_type=jnp.float32)
                      + bl2_ref[...])


def _final_layer(z2, acc2, cnt, batch2d, Wl1, bl1, Wl2, bl2):
    return pl.pallas_call(
        _final_body,
        grid=(NGRID,),
        in_specs=[pl.BlockSpec((NB, H2), lambda i: (i, 0))]
        + _acc_specs(H2) + _acc_specs(CW) + [
            pl.BlockSpec((NB, 1), lambda i: (i, 0)),
            pl.BlockSpec((H2, H1), lambda i: (0, 0)),
            pl.BlockSpec((1, H1), lambda i: (0, 0)),
            pl.BlockSpec((H1, C), lambda i: (0, 0)),
            pl.BlockSpec((1, C), lambda i: (0, 0)),
        ],
        out_specs=pl.BlockSpec((G, C), lambda i: (0, 0)),
        out_shape=jax.ShapeDtypeStruct((G, C), jnp.float32),
        scratch_shapes=[pltpu.VMEM((G, H2), jnp.float32)],
    )(z2, *([acc2] * R), *([cnt] * R), batch2d, Wl1, bl1, Wl2, bl2)


def kernel(x, edge_index, edge_attr, batch, W1, root1, b1, W2, root2, b2,
           Wl1, bl1, Wl2, bl2):
    src = edge_index[0].astype(jnp.int32)
    dst = edge_index[1].astype(jnp.int32)
    t = edge_attr[:, 0].astype(jnp.int32)
    gidx = t * N + src
    sidx = t * N + dst
    npad = EPAD - E
    spread = jnp.arange(npad, dtype=jnp.int32) % PAD_ROWS
    gidx = jnp.concatenate([gidx, spread]).reshape(NW, NBLK, EB)
    sidx = jnp.concatenate([sidx, RN + spread]).reshape(NW, NBLK, EB)

    zeros16 = jnp.zeros((RNP, H1), jnp.float32)
    zeros32 = jnp.zeros((RNP, H2), jnp.float32)
    zeros8 = jnp.zeros((RNP, CW), jnp.float32)
    ones8 = jnp.ones((EB, CW), jnp.float32)

    y1 = _relation_matmul(x, W1).reshape(RN, H1)
    acc1, cnt = _scatter1(y1, gidx, sidx, zeros16, zeros8, ones8)

    y2, z2 = _mid_layer(x, acc1, cnt, root1, b1.reshape(1, H1), W2, root2,
                        b2.reshape(1, H2))
    (acc2,) = _scatter2(y2.reshape(RN, H2), gidx, sidx, zeros32)

    batch2d = batch.astype(jnp.int32).reshape(N, 1)
    return _final_layer(z2, acc2, cnt, batch2d, Wl1, bl1.reshape(1, H1),
                        Wl2, bl2.reshape(1, C))


# speedup vs baseline: 1.7183x; 1.7183x over previous
import jax
import jax.numpy as jnp
import numpy as _np
from jax import lax
from jax.experimental import pallas as pl
from jax.experimental.pallas import tpu as pltpu
from jax.experimental.pallas import tpu_sc as plsc

N = 10000
E = 320000
D = 128
R = 4
H1 = 16
H2 = 32
C = 8
G = 64

RN = R * N
PAD_ROWS = 64
RNP = RN + PAD_ROWS
NSUB = 16
NCORE = 2
NW = NSUB * NCORE
EB = 128
EPW = 10240
NBLK = EPW // EB
NBUF = 5
EPAD = NW * EPW
SUB_ROWS = RNP // NSUB

LW = R * H2
RNP4 = RNP // R
NB = 2000
NGRID = N // NB

_selnp = _np.zeros((LW, LW), _np.float32)
for _j in range(LW):
    _selnp[(_j // H2) * H2 + H1, _j] = 1.0
_cvnp = _np.zeros((1, LW), _np.float32)
_cvnp[0, H1::H2] = 1.0


def _y1_body(x_ref, w_ref, cv_ref, y_ref):
    y_ref[...] = jnp.dot(x_ref[...], w_ref[...],
                         preferred_element_type=jnp.float32) + cv_ref[...]


def _relation_matmul(x, W1cat, cvec):
    return pl.pallas_call(
        _y1_body,
        grid=(NGRID,),
        in_specs=[pl.BlockSpec((NB, D), lambda i: (i, 0)),
                  pl.BlockSpec((D, LW), lambda i: (0, 0)),
                  pl.BlockSpec((1, LW), lambda i: (0, 0))],
        out_specs=pl.BlockSpec((NB, LW), lambda i: (i, 0)),
        out_shape=jax.ShapeDtypeStruct((N, LW), jnp.float32),
    )(x, W1cat, cvec)


def _make_edge_scatter(width, nbuf):
    mesh = plsc.VectorSubcoreMesh(core_axis_name="c", subcore_axis_name="s")
    out_type = [jax.ShapeDtypeStruct((NCORE, RNP, width), jnp.float32)]
    scratch = [
        pltpu.VMEM_SHARED((RNP, width), jnp.float32),
        pltpu.VMEM((NBLK, EB), jnp.int32),
        pltpu.VMEM((NBLK, EB), jnp.int32),
    ] + [pltpu.VMEM((EB, width), jnp.float32) for _ in range(nbuf)] + [
        pltpu.SemaphoreType.DMA((nbuf,)),
        pltpu.SemaphoreType.DMA((nbuf,)),
    ]

    def body(y_hbm, g_hbm, s_hbm, z_hbm, out_acc, acc_sh, g_all, s_all,
             *rest):
        rbuf = rest[:nbuf]
        gsem, ssem = rest[nbuf:]
        cid = lax.axis_index("c")
        sid = lax.axis_index("s")
        wid = cid * NSUB + sid
        rows = pl.ds(sid * SUB_ROWS, SUB_ROWS)
        pltpu.sync_copy(g_hbm.at[wid], g_all)
        pltpu.sync_copy(s_hbm.at[wid], s_all)
        pltpu.sync_copy(z_hbm.at[rows], acc_sh.at[rows])
        plsc.subcore_barrier()

        for k in range(nbuf):
            pltpu.async_copy(y_hbm.at[g_all.at[k]], rbuf[k], gsem.at[k])

        def step(j, carry):
            for k in range(nbuf):
                b = j * nbuf + k
                pltpu.make_async_copy(y_hbm.at[g_all.at[b]], rbuf[k],
                                      gsem.at[k]).wait()
                pltpu.async_copy(rbuf[k], acc_sh.at[s_all.at[b]],
                                 ssem.at[k], add=True)
            for k in range(nbuf):
                b = j * nbuf + k
                pltpu.make_async_copy(rbuf[k], acc_sh.at[s_all.at[b]],
                                      ssem.at[k]).wait()

                @pl.when(j < NBLK // nbuf - 1)
                def _(k=k, b=b):
                    pltpu.async_copy(y_hbm.at[g_all.at[b + nbuf]], rbuf[k],
                                     gsem.at[k])
            return carry

        lax.fori_loop(0, NBLK // nbuf, step, 0)
        plsc.subcore_barrier()
        pltpu.sync_copy(acc_sh.at[rows], out_acc.at[cid, rows])

    return pl.kernel(body, out_type=out_type, mesh=mesh, scratch_types=scratch,
                     compiler_params=pltpu.CompilerParams(
                         use_tc_tiling_on_sc=False))


_edge_scatter = _make_edge_scatter(H2, NBUF)


def _fold_msg(ap, sel):
    q = ap / jnp.maximum(jnp.dot(ap, sel, preferred_element_type=jnp.float32),
                         1.0)
    return (q[:, 0:H2] + q[:, H2:2 * H2] + q[:, 2 * H2:3 * H2]
            + q[:, 3 * H2:4 * H2])


def _mid_body(x_ref, a_ref, sel_ref, r1_ref, b1_ref, w2_ref, r2_ref, b2_ref,
              y2_ref, z2_ref):
    sel = sel_ref[...]
    msg = _fold_msg(a_ref[0] + a_ref[1], sel)
    h1 = jax.nn.relu(
        jnp.dot(x_ref[...], r1_ref[...], preferred_element_type=jnp.float32)
        + b1_ref[...] + msg)
    y2_ref[...] = jnp.dot(h1, w2_ref[...], preferred_element_type=jnp.float32)
    z2_ref[...] = (jnp.dot(h1, r2_ref[...], preferred_element_type=jnp.float32)
                   + b2_ref[...])


def _mid_layer(x, acc1p, sel, r1p, b1p, W2cat, r2cat, b2t):
    return pl.pallas_call(
        _mid_body,
        grid=(NGRID,),
        in_specs=[
            pl.BlockSpec((NB, D), lambda i: (i, 0)),
            pl.BlockSpec((NCORE, NB, LW), lambda i: (0, i, 0)),
            pl.BlockSpec((LW, LW), lambda i: (0, 0)),
            pl.BlockSpec((D, H2), lambda i: (0, 0)),
            pl.BlockSpec((1, H2), lambda i: (0, 0)),
            pl.BlockSpec((H2, LW), lambda i: (0, 0)),
            pl.BlockSpec((H2, LW), lambda i: (0, 0)),
            pl.BlockSpec((1, LW), lambda i: (0, 0)),
        ],
        out_specs=[
            pl.BlockSpec((NB, LW), lambda i: (i, 0)),
            pl.BlockSpec((NB, LW), lambda i: (i, 0)),
        ],
        out_shape=[
            jax.ShapeDtypeStruct((N, LW), jnp.float32),
            jax.ShapeDtypeStruct((N, LW), jnp.float32),
        ],
    )(x, acc1p, sel, r1p, b1p, W2cat, r2cat, b2t)


def _final_body(z_ref, a2_ref, a1_ref, sel_ref, b_ref, wl1_ref, bl1_ref,
                wl2_ref, bl2_ref, o_ref, p_ref):
    i = pl.program_id(0)
    sel = sel_ref[...]
    ap = a2_ref[0] + a2_ref[1]
    cb = jnp.dot(a1_ref[0] + a1_ref[1], sel,
                 preferred_element_type=jnp.float32)
    q = ap / jnp.maximum(cb, 1.0)
    msg = (q[:, 0:H2] + q[:, H2:2 * H2] + q[:, 2 * H2:3 * H2]
           + q[:, 3 * H2:4 * H2])
    h2 = jax.nn.relu(z_ref[:, 0:H2] + msg)

    @pl.when(i == 0)
    def _():
        p_ref[...] = jnp.zeros((G, H2), jnp.float32)

    b = b_ref[...]
    gmin = b_ref[0, 0]
    gmax = b_ref[NB - 1, 0]

    def pool(g, carry):
        v = jnp.max(jnp.where(b == g, h2, 0.0), axis=0, keepdims=True)
        p_ref[pl.ds(g, 1), :] = jnp.maximum(p_ref[pl.ds(g, 1), :], v)
        return carry

    lax.fori_loop(gmin, gmax + 1, pool, 0)

    @pl.when(i == NGRID - 1)
    def _():
        p = p_ref[...]
        hh = jax.nn.relu(
            jnp.dot(p, wl1_ref[...], preferred_element_type=jnp.float32)
            + bl1_ref[...])
        o_ref[...] = (jnp.dot(hh, wl2_ref[...], preferred_element_type=jnp.float32)
                      + bl2_ref[...])


def _final_layer(z2, acc2p, acc1p, sel, batch2d, Wl1, bl1, Wl2, bl2):
    return pl.pallas_call(
        _final_body,
        grid=(NGRID,),
        in_specs=[
            pl.BlockSpec((NB, LW), lambda i: (i, 0)),
            pl.BlockSpec((NCORE, NB, LW), lambda i: (0, i, 0)),
            pl.BlockSpec((NCORE, NB, LW), lambda i: (0, i, 0)),
            pl.BlockSpec((LW, LW), lambda i: (0, 0)),
            pl.BlockSpec((NB, 1), lambda i: (i, 0)),
            pl.BlockSpec((H2, H1), lambda i: (0, 0)),
            pl.BlockSpec((1, H1), lambda i: (0, 0)),
            pl.BlockSpec((H1, C), lambda i: (0, 0)),
            pl.BlockSpec((1, C), lambda i: (0, 0)),
        ],
        out_specs=pl.BlockSpec((G, C), lambda i: (0, 0)),
        out_shape=jax.ShapeDtypeStruct((G, C), jnp.float32),
        scratch_shapes=[pltpu.VMEM((G, H2), jnp.float32)],
    )(z2, acc2p, acc1p, sel, batch2d, Wl1, bl1, Wl2, bl2)


def kernel(x, edge_index, edge_attr, batch, W1, root1, b1, W2, root2, b2,
           Wl1, bl1, Wl2, bl2):
    src = edge_index[0].astype(jnp.int32)
    dst = edge_index[1].astype(jnp.int32)
    t = edge_attr[:, 0].astype(jnp.int32)
    gidx = src * R + t
    sidx = dst * R + t
    npad = EPAD - E
    spread = jnp.arange(npad, dtype=jnp.int32) % PAD_ROWS
    gidx = jnp.concatenate([gidx, spread]).reshape(NW, NBLK, EB)
    sidx = jnp.concatenate([sidx, RN + spread]).reshape(NW, NBLK, EB)

    zeros32 = jnp.zeros((RNP, H2), jnp.float32)
    sel = jnp.asarray(_selnp)
    cvec = jnp.asarray(_cvnp)

    zc = jnp.zeros((D, H2 - H1), jnp.float32)
    W1cat = jnp.concatenate(
        [jnp.concatenate([W1[r], zc], axis=-1) for r in range(R)], axis=-1)
    r1p = jnp.concatenate([root1, zc], axis=-1)
    b1p = jnp.concatenate([b1, jnp.zeros((H1,), jnp.float32)]).reshape(1, H2)
    zr = jnp.zeros((H2 - H1, H2), jnp.float32)
    W2cat = jnp.concatenate(
        [jnp.concatenate([W2[r], zr], axis=0) for r in range(R)], axis=-1)
    r2cat = jnp.tile(jnp.concatenate([root2, zr], axis=0), (1, R))
    b2t = jnp.tile(b2, R).reshape(1, LW)

    y1 = _relation_matmul(x, W1cat, cvec).reshape(RN, H2)
    (acc1,) = _edge_scatter(y1, gidx, sidx, zeros32)
    acc1p = acc1.reshape(NCORE, RNP4, LW)

    y2, z2 = _mid_layer(x, acc1p, sel, r1p, b1p, W2cat, r2cat, b2t)
    (acc2,) = _edge_scatter(y2.reshape(RN, H2), gidx, sidx, zeros32)
    acc2p = acc2.reshape(NCORE, RNP4, LW)

    batch2d = batch.astype(jnp.int32).reshape(N, 1)
    return _final_layer(z2, acc2p, acc1p, sel, batch2d, Wl1,
                        bl1.reshape(1, H1), Wl2, bl2.reshape(1, C))


# speedup vs baseline: 1.7924x; 1.0431x over previous
import jax
import jax.numpy as jnp
import numpy as _np
from jax import lax
from jax.experimental import pallas as pl
from jax.experimental.pallas import tpu as pltpu
from jax.experimental.pallas import tpu_sc as plsc

N = 10000
E = 320000
D = 128
R = 4
H1 = 16
H2 = 32
C = 8
G = 64

RN = R * N
PAD_ROWS = 64
RNP = RN + PAD_ROWS
NSUB = 16
NCORE = 2
NW = NSUB * NCORE
EB = 128
EPW = 10240
NBLK = EPW // EB
NBUF = 5
EPAD = NW * EPW
SUB_ROWS = RNP // NSUB

LW = R * H2
RNP4 = RNP // R
NB = 2000
NGRID = N // NB

_selnp = _np.zeros((LW, LW), _np.float32)
for _j in range(LW):
    _selnp[(_j // H2) * H2 + H1, _j] = 1.0
_cvnp = _np.zeros((1, LW), _np.float32)
_cvnp[0, H1::H2] = 1.0


EROWS = E // EB
TROWS = EPAD // EB
PADB = TROWS - EROWS


def _idx_body(ei_ref, t_ref, g_ref, s_ref):
    t = t_ref[...]
    row = lax.broadcasted_iota(jnp.int32, (PADB, EB), 0)
    lane = lax.broadcasted_iota(jnp.int32, (PADB, EB), 1)
    pad = (row * EB + lane) % PAD_ROWS
    g_ref[...] = jnp.concatenate([ei_ref[0] * R + t, pad], axis=0)
    s_ref[...] = jnp.concatenate([ei_ref[1] * R + t, RN + pad], axis=0)


def _edge_indices(ei3, t3):
    return pl.pallas_call(
        _idx_body,
        grid=(1,),
        in_specs=[pl.BlockSpec((2, EROWS, EB), lambda i: (0, 0, 0)),
                  pl.BlockSpec((EROWS, EB), lambda i: (0, 0))],
        out_specs=[pl.BlockSpec((TROWS, EB), lambda i: (0, 0)),
                   pl.BlockSpec((TROWS, EB), lambda i: (0, 0))],
        out_shape=[jax.ShapeDtypeStruct((TROWS, EB), jnp.int32),
                   jax.ShapeDtypeStruct((TROWS, EB), jnp.int32)],
    )(ei3, t3)


def _y1_body(x_ref, w_ref, cv_ref, y_ref):
    y_ref[...] = jnp.dot(x_ref[...], w_ref[...],
                         preferred_element_type=jnp.float32) + cv_ref[...]


def _relation_matmul(x, W1cat, cvec):
    return pl.pallas_call(
        _y1_body,
        grid=(NGRID,),
        in_specs=[pl.BlockSpec((NB, D), lambda i: (i, 0)),
                  pl.BlockSpec((D, LW), lambda i: (0, 0)),
                  pl.BlockSpec((1, LW), lambda i: (0, 0))],
        out_specs=pl.BlockSpec((NB, LW), lambda i: (i, 0)),
        out_shape=jax.ShapeDtypeStruct((N, LW), jnp.float32),
    )(x, W1cat, cvec)


def _make_edge_scatter(width, nbuf):
    mesh = plsc.VectorSubcoreMesh(core_axis_name="c", subcore_axis_name="s")
    out_type = [jax.ShapeDtypeStruct((NCORE, RNP, width), jnp.float32)]
    scratch = [
        pltpu.VMEM_SHARED((RNP, width), jnp.float32),
        pltpu.VMEM((NBLK, EB), jnp.int32),
        pltpu.VMEM((NBLK, EB), jnp.int32),
    ] + [pltpu.VMEM((EB, width), jnp.float32) for _ in range(nbuf)] + [
        pltpu.SemaphoreType.DMA((nbuf,)),
        pltpu.SemaphoreType.DMA((nbuf,)),
    ]

    def body(y_hbm, g_hbm, s_hbm, z_hbm, out_acc, acc_sh, g_all, s_all,
             *rest):
        rbuf = rest[:nbuf]
        gsem, ssem = rest[nbuf:]
        cid = lax.axis_index("c")
        sid = lax.axis_index("s")
        wid = cid * NSUB + sid
        rows = pl.ds(sid * SUB_ROWS, SUB_ROWS)
        pltpu.sync_copy(g_hbm.at[wid], g_all)
        pltpu.sync_copy(s_hbm.at[wid], s_all)
        pltpu.sync_copy(z_hbm.at[rows], acc_sh.at[rows])
        plsc.subcore_barrier()

        for k in range(nbuf):
            pltpu.async_copy(y_hbm.at[g_all.at[k]], rbuf[k], gsem.at[k])

        def step(j, carry):
            for k in range(nbuf):
                b = j * nbuf + k
                pltpu.make_async_copy(y_hbm.at[g_all.at[b]], rbuf[k],
                                      gsem.at[k]).wait()
                pltpu.async_copy(rbuf[k], acc_sh.at[s_all.at[b]],
                                 ssem.at[k], add=True)
            for k in range(nbuf):
                b = j * nbuf + k
                pltpu.make_async_copy(rbuf[k], acc_sh.at[s_all.at[b]],
                                      ssem.at[k]).wait()

                @pl.when(j < NBLK // nbuf - 1)
                def _(k=k, b=b):
                    pltpu.async_copy(y_hbm.at[g_all.at[b + nbuf]], rbuf[k],
                                     gsem.at[k])
            return carry

        lax.fori_loop(0, NBLK // nbuf, step, 0)
        plsc.subcore_barrier()
        pltpu.sync_copy(acc_sh.at[rows], out_acc.at[cid, rows])

    return pl.kernel(body, out_type=out_type, mesh=mesh, scratch_types=scratch,
                     compiler_params=pltpu.CompilerParams(
                         use_tc_tiling_on_sc=False))


_edge_scatter = _make_edge_scatter(H2, NBUF)


def _fold_msg(ap, sel):
    q = ap / jnp.maximum(jnp.dot(ap, sel, preferred_element_type=jnp.float32),
                         1.0)
    return (q[:, 0:H2] + q[:, H2:2 * H2] + q[:, 2 * H2:3 * H2]
            + q[:, 3 * H2:4 * H2])


def _mid_body(x_ref, a_ref, sel_ref, r1_ref, b1_ref, w2_ref, r2_ref, b2_ref,
              y2_ref, z2_ref):
    sel = sel_ref[...]
    msg = _fold_msg(a_ref[0] + a_ref[1], sel)
    h1 = jax.nn.relu(
        jnp.dot(x_ref[...], r1_ref[...], preferred_element_type=jnp.float32)
        + b1_ref[...] + msg)
    y2_ref[...] = jnp.dot(h1, w2_ref[...], preferred_element_type=jnp.float32)
    z2_ref[...] = (jnp.dot(h1, r2_ref[...], preferred_element_type=jnp.float32)
                   + b2_ref[...])


def _mid_layer(x, acc1p, sel, r1p, b1p, W2cat, r2cat, b2t):
    return pl.pallas_call(
        _mid_body,
        grid=(NGRID,),
        in_specs=[
            pl.BlockSpec((NB, D), lambda i: (i, 0)),
            pl.BlockSpec((NCORE, NB, LW), lambda i: (0, i, 0)),
            pl.BlockSpec((LW, LW), lambda i: (0, 0)),
            pl.BlockSpec((D, H2), lambda i: (0, 0)),
            pl.BlockSpec((1, H2), lambda i: (0, 0)),
            pl.BlockSpec((H2, LW), lambda i: (0, 0)),
            pl.BlockSpec((H2, LW), lambda i: (0, 0)),
            pl.BlockSpec((1, LW), lambda i: (0, 0)),
        ],
        out_specs=[
            pl.BlockSpec((NB, LW), lambda i: (i, 0)),
            pl.BlockSpec((NB, LW), lambda i: (i, 0)),
        ],
        out_shape=[
            jax.ShapeDtypeStruct((N, LW), jnp.float32),
            jax.ShapeDtypeStruct((N, LW), jnp.float32),
        ],
    )(x, acc1p, sel, r1p, b1p, W2cat, r2cat, b2t)


def _final_body(z_ref, a2_ref, a1_ref, sel_ref, b_ref, wl1_ref, bl1_ref,
                wl2_ref, bl2_ref, o_ref, p_ref):
    i = pl.program_id(0)
    sel = sel_ref[...]
    ap = a2_ref[0] + a2_ref[1]
    cb = jnp.dot(a1_ref[0] + a1_ref[1], sel,
                 preferred_element_type=jnp.float32)
    q = ap / jnp.maximum(cb, 1.0)
    msg = (q[:, 0:H2] + q[:, H2:2 * H2] + q[:, 2 * H2:3 * H2]
           + q[:, 3 * H2:4 * H2])
    h2 = jax.nn.relu(z_ref[:, 0:H2] + msg)

    @pl.when(i == 0)
    def _():
        p_ref[...] = jnp.zeros((G, H2), jnp.float32)

    b = b_ref[...]
    gmin = b_ref[0, 0]
    gmax = b_ref[NB - 1, 0]

    def pool(g, carry):
        v = jnp.max(jnp.where(b == g, h2, 0.0), axis=0, keepdims=True)
        p_ref[pl.ds(g, 1), :] = jnp.maximum(p_ref[pl.ds(g, 1), :], v)
        return carry

    lax.fori_loop(gmin, gmax + 1, pool, 0)

    @pl.when(i == NGRID - 1)
    def _():
        p = p_ref[...]
        hh = jax.nn.relu(
            jnp.dot(p, wl1_ref[...], preferred_element_type=jnp.float32)
            + bl1_ref[...])
        o_ref[...] = (jnp.dot(hh, wl2_ref[...], preferred_element_type=jnp.float32)
                      + bl2_ref[...])


def _final_layer(z2, acc2p, acc1p, sel, batch2d, Wl1, bl1, Wl2, bl2):
    return pl.pallas_call(
        _final_body,
        grid=(NGRID,),
        in_specs=[
            pl.BlockSpec((NB, LW), lambda i: (i, 0)),
            pl.BlockSpec((NCORE, NB, LW), lambda i: (0, i, 0)),
            pl.BlockSpec((NCORE, NB, LW), lambda i: (0, i, 0)),
            pl.BlockSpec((LW, LW), lambda i: (0, 0)),
            pl.BlockSpec((NB, 1), lambda i: (i, 0)),
            pl.BlockSpec((H2, H1), lambda i: (0, 0)),
            pl.BlockSpec((1, H1), lambda i: (0, 0)),
            pl.BlockSpec((H1, C), lambda i: (0, 0)),
            pl.BlockSpec((1, C), lambda i: (0, 0)),
        ],
        out_specs=pl.BlockSpec((G, C), lambda i: (0, 0)),
        out_shape=jax.ShapeDtypeStruct((G, C), jnp.float32),
        scratch_shapes=[pltpu.VMEM((G, H2), jnp.float32)],
    )(z2, acc2p, acc1p, sel, batch2d, Wl1, bl1, Wl2, bl2)


def kernel(x, edge_index, edge_attr, batch, W1, root1, b1, W2, root2, b2,
           Wl1, bl1, Wl2, bl2):
    ei3 = edge_index.astype(jnp.int32).reshape(2, EROWS, EB)
    t3 = edge_attr.astype(jnp.int32).reshape(EROWS, EB)
    g3, s3 = _edge_indices(ei3, t3)
    gidx = g3.reshape(NW, NBLK, EB)
    sidx = s3.reshape(NW, NBLK, EB)

    zeros32 = jnp.zeros((RNP, H2), jnp.float32)
    sel = jnp.asarray(_selnp)
    cvec = jnp.asarray(_cvnp)

    zc = jnp.zeros((D, H2 - H1), jnp.float32)
    W1cat = jnp.concatenate(
        [jnp.concatenate([W1[r], zc], axis=-1) for r in range(R)], axis=-1)
    r1p = jnp.concatenate([root1, zc], axis=-1)
    b1p = jnp.concatenate([b1, jnp.zeros((H1,), jnp.float32)]).reshape(1, H2)
    zr = jnp.zeros((H2 - H1, H2), jnp.float32)
    W2cat = jnp.concatenate(
        [jnp.concatenate([W2[r], zr], axis=0) for r in range(R)], axis=-1)
    r2cat = jnp.tile(jnp.concatenate([root2, zr], axis=0), (1, R))
    b2t = jnp.tile(b2, R).reshape(1, LW)

    y1 = _relation_matmul(x, W1cat, cvec).reshape(RN, H2)
    (acc1,) = _edge_scatter(y1, gidx, sidx, zeros32)
    acc1p = acc1.reshape(NCORE, RNP4, LW)

    y2, z2 = _mid_layer(x, acc1p, sel, r1p, b1p, W2cat, r2cat, b2t)
    (acc2,) = _edge_scatter(y2.reshape(RN, H2), gidx, sidx, zeros32)
    acc2p = acc2.reshape(NCORE, RNP4, LW)

    batch2d = batch.astype(jnp.int32).reshape(N, 1)
    return _final_layer(z2, acc2p, acc1p, sel, batch2d, Wl1,
                        bl1.reshape(1, H1), Wl2, bl2.reshape(1, C))
